# baseline (speedup 1.0000x reference)
import jax
import jax.numpy as jnp
from jax.experimental import pallas as pl

N = 512
K = 81
BI = 8


def _body(psj, p0j, p1j, p2j, psi, p0i, p1i, p2i, sv, sv0, sv1, sv2, cut,
          out_ref):
    disp = (psj[...][None, :, :] - psi[...][:, None, :]) + sv[...][0][None, None, :]
    d0 = (p0j[...][None, :, :] - p0i[...][:, None, :]) + sv0[...][0][None, None, :]
    d1 = (p1j[...][None, :, :] - p1i[...][:, None, :]) + sv1[...][0][None, None, :]
    d2 = (p2j[...][None, :, :] - p2i[...][:, None, :]) + sv2[...][0][None, None, :]
    rs = jnp.sqrt((d0 * d0 + d1 * d1) + d2 * d2)
    mask = (rs <= cut[...][:, :, None]) & (rs > 1e-8)
    out_ref[...] = jnp.where(mask, disp, 0.0)


def _field(PS, P0, P1, P2, SV, SV0, SV1, SV2, CUT):
    full = pl.BlockSpec((N, K), lambda i: (0, 0))
    rows = pl.BlockSpec((BI, K), lambda i: (i, 0))
    one = pl.BlockSpec((1, K), lambda i: (0, 0))
    return pl.pallas_call(
        _body,
        grid=(N // BI,),
        in_specs=[full, full, full, full, rows, rows, rows, rows,
                  one, one, one, one,
                  pl.BlockSpec((BI, N), lambda i: (i, 0))],
        out_specs=pl.BlockSpec((BI, N, K), lambda i: (i, 0, 0)),
        out_shape=jax.ShapeDtypeStruct((N, N, K), jnp.float32),
    )(PS, P0, P1, P2, PS, P0, P1, P2, SV, SV0, SV1, SV2, CUT)


def kernel(positions, cell, radii_table, numbers):
    positions = positions.astype(jnp.float32)
    s = jnp.arange(-1, 2, dtype=positions.dtype)
    g = jnp.meshgrid(s, s, s, indexing="ij")
    shifts = jnp.stack(g, axis=-1).reshape(-1, 3)
    shift_vecs = shifts @ cell

    radii = jnp.take(radii_table, numbers, axis=0).astype(jnp.float32)
    gcut = 2.0 * jnp.max(radii)
    CUT = jnp.minimum(radii[:, None] + radii[None, :], gcut)

    PS = jnp.tile(positions, (1, 27))
    P0 = jnp.broadcast_to(positions[:, 0:1], (N, K))
    P1 = jnp.broadcast_to(positions[:, 1:2], (N, K))
    P2 = jnp.broadcast_to(positions[:, 2:3], (N, K))
    SV = shift_vecs.reshape(1, K)
    SV0 = jnp.repeat(shift_vecs[:, 0], 3).reshape(1, K)
    SV1 = jnp.repeat(shift_vecs[:, 1], 3).reshape(1, K)
    SV2 = jnp.repeat(shift_vecs[:, 2], 3).reshape(1, K)

    out81 = _field(PS, P0, P1, P2, SV, SV0, SV1, SV2, CUT)
    return out81.reshape(N, N, 27, 3)
